# baseline (speedup 1.0000x reference)
import functools

import jax
import jax.numpy as jnp
from jax import lax
from jax.experimental import pallas as pl
from jax.experimental.pallas import tpu as pltpu
from jax.experimental.pallas import tpu_sc as plsc

D = 64
DP = 72
K = 5
L = 16

_info = plsc.get_sparse_core_info()
NC, NS = _info.num_cores, _info.num_subcores
NW = NC * NS


DW = D // 2


def _sc_scores(batch: int):
    bpw = batch // NW
    chunk = 64
    nchunk = bpw // chunk
    rows_per_chunk = chunk * (2 + K)
    mesh = plsc.VectorSubcoreMesh(core_axis_name="c", subcore_axis_name="s")

    @functools.partial(
        pl.kernel,
        out_type=jax.ShapeDtypeStruct((2, NW * L), jnp.float32),
        mesh=mesh,
        compiler_params=pltpu.CompilerParams(
            needs_layout_passes=False, use_tc_tiling_on_sc=False
        ),
        scratch_types=[
            pltpu.VMEM((nchunk, rows_per_chunk), jnp.int32),
            pltpu.VMEM((nchunk, chunk, K), jnp.int32),
            pltpu.VMEM((rows_per_chunk, DP), jnp.float32),
            pltpu.VMEM((rows_per_chunk, DP), jnp.float32),
            pltpu.VMEM((2, L), jnp.float32),
            pltpu.SemaphoreType.DMA,
            pltpu.SemaphoreType.DMA,
            pltpu.SemaphoreType.DMA,
        ],
    )
    def k(nodes_hbm, pos_hbm, neg_hbm, table_hbm, out_hbm,
          idx_all, negraw, rows_a, rows_b, partials, sem_i, sem_e, sem_o):
        wid = lax.axis_index("s") * NC + lax.axis_index("c")
        base0 = wid * bpw
        iot = lax.iota(jnp.int32, L)
        bufs = (rows_a, rows_b)
        ssems = (sem_e, sem_o)

        def softplus(x):
            e = jnp.exp(-jnp.abs(x))
            e1 = 1.0 + e
            t = e * (6.0 + e) / (6.0 + 4.0 * e)
            t = t - 1.0 + e1 * jnp.exp(-t)
            t = t - 1.0 + e1 * jnp.exp(-t)
            return jnp.maximum(x, 0.0) + t

        idx_cps = []
        for c in range(nchunk):
            bc = base0 + c * chunk
            idx_cps.append(pltpu.async_copy(
                nodes_hbm.at[pl.ds(bc, chunk)],
                idx_all.at[c, pl.ds(0, chunk)], sem_i))
            idx_cps.append(pltpu.async_copy(
                pos_hbm.at[pl.ds(bc, chunk)],
                idx_all.at[c, pl.ds(chunk, chunk)], sem_i))
            idx_cps.append(pltpu.async_copy(
                neg_hbm.at[pl.ds(bc, chunk), :], negraw.at[c], sem_i))
        for cp in idx_cps:
            cp.wait()

        for c in range(nchunk):
            for j in range(chunk * K // L):
                f = j * L + iot
                vals = plsc.load_gather(negraw, [jnp.full((L,), c, jnp.int32),
                                                f // K, f % K])
                idx_all[c, pl.ds(2 * chunk + j * L, L)] = vals

        streams = [None] * nchunk
        streams[0] = pltpu.async_copy(
            table_hbm.at[idx_all.at[0]], bufs[0], ssems[0])

        zero = jnp.zeros((L,), jnp.float32)
        pp, nn = zero, zero
        for c in range(nchunk):
            if c + 1 < nchunk:
                streams[c + 1] = pltpu.async_copy(
                    table_hbm.at[idx_all.at[c + 1]],
                    bufs[(c + 1) % 2], ssems[(c + 1) % 2])
            streams[c].wait()
            rows = bufs[c % 2]

            def group_body(g, carry, rows=rows):
                pp, nn = carry
                rb = g * L + iot
                rp = rb + chunk
                rg = 2 * chunk + rb * K
                accs = [zero] * (1 + K)

                def d_body(d4, accs):
                    accs = list(accs)
                    for dd in range(4):
                        col = jnp.bitwise_and(iot + (d4 * 4 + dd), D - 1)
                        nd = plsc.load_gather(rows, [rb, col])
                        pd = plsc.load_gather(rows, [rp, col])
                        accs[0] = accs[0] + nd * pd
                        for kk in range(K):
                            gd = plsc.load_gather(rows, [rg + kk, col])
                            accs[1 + kk] = accs[1 + kk] + nd * gd
                    return tuple(accs)

                accs = lax.fori_loop(0, D // 4, d_body, tuple(accs))
                pp = pp + softplus(-accs[0])
                for kk in range(K):
                    nn = nn + softplus(accs[1 + kk])
                return pp, nn

            pp, nn = lax.fori_loop(0, chunk // L, group_body, (pp, nn))

        partials[0, :] = pp
        partials[1, :] = nn
        pltpu.sync_copy(partials, out_hbm.at[:, pl.ds(wid * L, L)])

    return k


def _tc_relayout(v: int):
    blk = 1024
    grid = (v + blk - 1) // blk

    def body(tin, tout):
        tout[:, 0:D] = tin[...].T

    return pl.pallas_call(
        body,
        grid=(grid,),
        in_specs=[pl.BlockSpec((D, blk), lambda i: (0, i))],
        out_specs=pl.BlockSpec((blk, 128), lambda i: (i, 0)),
        out_shape=jax.ShapeDtypeStruct((v, 128), jnp.float32),
    )


def _tc_losses(batch: int):
    def body(s_ref, pos_out, neg_out):
        s = s_ref[...]
        pos_out[0, 0] = jnp.sum(s[0:1, :]) / batch
        neg_out[0, 0] = jnp.sum(s[1:2, :]) / (batch * K)

    return pl.pallas_call(
        body,
        out_shape=[
            jax.ShapeDtypeStruct((1, 1), jnp.float32),
            jax.ShapeDtypeStruct((1, 1), jnp.float32),
        ],
        out_specs=[
            pl.BlockSpec(memory_space=pltpu.SMEM),
            pl.BlockSpec(memory_space=pltpu.SMEM),
        ],
    )


def kernel(nodes, pos_neighbors, neg_neighbors, node_embeddings):
    batch = nodes.shape[0]
    table_p = jnp.pad(node_embeddings, ((0, 0), (0, DP - D)))
    partials = _sc_scores(batch)(
        nodes, pos_neighbors, neg_neighbors, table_p)
    pos_loss, neg_loss = _tc_losses(batch)(partials)
    return (pos_loss[0, 0], neg_loss[0, 0])


# speedup vs baseline: 1.2633x; 1.2633x over previous
import functools

import jax
import jax.numpy as jnp
from jax import lax
from jax.experimental import pallas as pl
from jax.experimental.pallas import tpu as pltpu
from jax.experimental.pallas import tpu_sc as plsc

D = 64
DP = 128
K = 5
L = 16

_info = plsc.get_sparse_core_info()
NC, NS = _info.num_cores, _info.num_subcores
NW = NC * NS


DW = D // 2


def _sc_scores(batch: int, v2: int):
    bpw = batch // NW
    chunk = 32
    nchunk = bpw // chunk
    rows_per_chunk = chunk * (2 + K)
    mesh = plsc.VectorSubcoreMesh(core_axis_name="c", subcore_axis_name="s")

    @functools.partial(
        pl.kernel,
        out_type=jax.ShapeDtypeStruct((2, NW * L), jnp.float32),
        mesh=mesh,
        compiler_params=pltpu.CompilerParams(
            needs_layout_passes=False, use_tc_tiling_on_sc=False
        ),
        scratch_types=[
            pltpu.VMEM((nchunk, rows_per_chunk), jnp.int32),
            pltpu.VMEM((nchunk, rows_per_chunk), jnp.int32),
            pltpu.VMEM((nchunk, rows_per_chunk), jnp.int32),
            pltpu.VMEM((nchunk, chunk, K), jnp.int32),
            pltpu.VMEM((rows_per_chunk, DP), jnp.float32),
            pltpu.VMEM((rows_per_chunk, DP), jnp.float32),
            pltpu.VMEM((2, L), jnp.float32),
            pltpu.SemaphoreType.DMA,
            pltpu.SemaphoreType.DMA,
            pltpu.SemaphoreType.DMA,
            pltpu.SemaphoreType.DMA,
            pltpu.SemaphoreType.DMA,
        ],
    )
    def k(nodes_hbm, pos_hbm, neg_hbm, tab_lo, tab_hi, out_hbm,
          idx_all, idx_lo, idx_hi, negraw, rows_a, rows_b, partials,
          sem_i, sem_ae, sem_ao, sem_be, sem_bo):
        wid = lax.axis_index("s") * NC + lax.axis_index("c")
        base0 = wid * bpw
        iot = lax.iota(jnp.int32, L)
        bufs = (rows_a, rows_b)
        asems = (sem_ae, sem_ao)
        bsems = (sem_be, sem_bo)

        def softplus(x):
            e = jnp.exp(-jnp.abs(x))
            e1 = 1.0 + e
            t = e * (6.0 + e) / (6.0 + 4.0 * e)
            t = t - 1.0 + e1 * jnp.exp(-t)
            t = t - 1.0 + e1 * jnp.exp(-t)
            return jnp.maximum(x, 0.0) + t

        idx_cps = []
        for c in range(nchunk):
            bc = base0 + c * chunk
            idx_cps.append(pltpu.async_copy(
                nodes_hbm.at[pl.ds(bc, chunk)],
                idx_all.at[c, pl.ds(0, chunk)], sem_i))
            idx_cps.append(pltpu.async_copy(
                pos_hbm.at[pl.ds(bc, chunk)],
                idx_all.at[c, pl.ds(chunk, chunk)], sem_i))
            idx_cps.append(pltpu.async_copy(
                neg_hbm.at[pl.ds(bc, chunk), :], negraw.at[c], sem_i))
        for cp in idx_cps:
            cp.wait()

        for c in range(nchunk):
            for j in range(chunk * K // L):
                f = j * L + iot
                vals = plsc.load_gather(negraw, [jnp.full((L,), c, jnp.int32),
                                                f // K, f % K])
                idx_all[c, pl.ds(2 * chunk + j * L, L)] = vals

        for c in range(nchunk):
            for j in range(rows_per_chunk // L):
                vals = idx_all[c, pl.ds(j * L, L)]
                idx_lo[c, pl.ds(j * L, L)] = jnp.where(vals < v2, vals, -1)
                idx_hi[c, pl.ds(j * L, L)] = jnp.where(vals >= v2, vals - v2, -1)

        def issue(c):
            buf = bufs[c % 2]
            return (
                pltpu.async_copy(
                    tab_lo.at[plsc.Indices(idx_lo.at[c], ignored_value=-1)],
                    buf, asems[c % 2]),
                pltpu.async_copy(
                    tab_hi.at[plsc.Indices(idx_hi.at[c], ignored_value=-1)],
                    buf, bsems[c % 2]),
            )

        streams = [None] * nchunk
        streams[0] = issue(0)

        zero = jnp.zeros((L,), jnp.float32)
        pp, nn = zero, zero
        for c in range(nchunk):
            if c + 1 < nchunk:
                streams[c + 1] = issue(c + 1)
            streams[c][0].wait()
            streams[c][1].wait()
            rows = bufs[c % 2]

            def group_body(g, carry, rows=rows):
                pp, nn = carry
                rb = g * L + iot
                rp = rb + chunk
                rg = 2 * chunk + rb * K
                accs = [zero] * (1 + K)

                def d_body(d4, accs):
                    accs = list(accs)
                    for dd in range(4):
                        col = jnp.bitwise_and(iot + (d4 * 4 + dd), D - 1)
                        nd = plsc.load_gather(rows, [rb, col])
                        pd = plsc.load_gather(rows, [rp, col])
                        accs[0] = accs[0] + nd * pd
                        for kk in range(K):
                            gd = plsc.load_gather(rows, [rg + kk, col])
                            accs[1 + kk] = accs[1 + kk] + nd * gd
                    return tuple(accs)

                accs = lax.fori_loop(0, D // 4, d_body, tuple(accs))
                pp = pp + softplus(-accs[0])
                for kk in range(K):
                    nn = nn + softplus(accs[1 + kk])
                return pp, nn

            pp, nn = lax.fori_loop(0, chunk // L, group_body, (pp, nn))

        partials[0, :] = pp
        partials[1, :] = nn
        pltpu.sync_copy(partials, out_hbm.at[:, pl.ds(wid * L, L)])

    return k


def _tc_relayout(v: int):
    blk = 1024
    grid = (v + blk - 1) // blk

    def body(tin, tout):
        tout[:, 0:D] = tin[...].T

    return pl.pallas_call(
        body,
        grid=(grid,),
        in_specs=[pl.BlockSpec((D, blk), lambda i: (0, i))],
        out_specs=pl.BlockSpec((blk, 128), lambda i: (i, 0)),
        out_shape=jax.ShapeDtypeStruct((v, 128), jnp.float32),
    )


def _tc_losses(batch: int):
    def body(s_ref, pos_out, neg_out):
        s = s_ref[...]
        pos_out[0, 0] = jnp.sum(s[0:1, :]) / batch
        neg_out[0, 0] = jnp.sum(s[1:2, :]) / (batch * K)

    return pl.pallas_call(
        body,
        out_shape=[
            jax.ShapeDtypeStruct((1, 1), jnp.float32),
            jax.ShapeDtypeStruct((1, 1), jnp.float32),
        ],
        out_specs=[
            pl.BlockSpec(memory_space=pltpu.SMEM),
            pl.BlockSpec(memory_space=pltpu.SMEM),
        ],
    )


def kernel(nodes, pos_neighbors, neg_neighbors, node_embeddings):
    batch = nodes.shape[0]
    v = node_embeddings.shape[0]
    v2 = v // 2
    tab_lo = jnp.pad(node_embeddings[:v2], ((0, 0), (0, DP - D)))
    tab_hi = jnp.pad(node_embeddings[v2:], ((0, 0), (0, DP - D)))
    partials = _sc_scores(batch, v2)(
        nodes, pos_neighbors, neg_neighbors, tab_lo, tab_hi)
    pos_loss, neg_loss = _tc_losses(batch)(partials)
    return (pos_loss[0, 0], neg_loss[0, 0])


# speedup vs baseline: 1.8909x; 1.4967x over previous
import functools

import jax
import jax.numpy as jnp
from jax import lax
from jax.experimental import pallas as pl
from jax.experimental.pallas import tpu as pltpu
from jax.experimental.pallas import tpu_sc as plsc

D = 64
DP = 128
K = 5
L = 16

_info = plsc.get_sparse_core_info()
NC, NS = _info.num_cores, _info.num_subcores
NW = NC * NS


def _sc_scores(batch: int):
    bpw = batch // NW
    chunk = 32
    nchunk = bpw // chunk
    rows_per_chunk = chunk * (2 + K)
    mesh = plsc.VectorSubcoreMesh(core_axis_name="c", subcore_axis_name="s")

    @functools.partial(
        pl.kernel,
        out_type=jax.ShapeDtypeStruct((2, NW * L), jnp.float32),
        mesh=mesh,
        compiler_params=pltpu.CompilerParams(
            needs_layout_passes=False, use_tc_tiling_on_sc=False
        ),
        scratch_types=[
            pltpu.VMEM((nchunk, rows_per_chunk), jnp.int32),
            pltpu.VMEM((nchunk, chunk, K), jnp.int32),
            pltpu.VMEM((rows_per_chunk, DP), jnp.float32),
            pltpu.VMEM((rows_per_chunk, DP), jnp.float32),
            pltpu.VMEM((2, L), jnp.float32),
            pltpu.SemaphoreType.DMA,
            pltpu.SemaphoreType.DMA,
            pltpu.SemaphoreType.DMA,
        ],
    )
    def k(nodes_hbm, pos_hbm, neg_hbm, table_hbm, out_hbm,
          idx_all, negraw, rows_a, rows_b, partials, sem_i, sem_e, sem_o):
        wid = lax.axis_index("s") * NC + lax.axis_index("c")
        base0 = wid * bpw
        iot = lax.iota(jnp.int32, L)
        bufs = (rows_a, rows_b)
        ssems = (sem_e, sem_o)

        def softplus(x):
            e = jnp.exp(-jnp.abs(x))
            e1 = 1.0 + e
            t = e * (6.0 + e) / (6.0 + 4.0 * e)
            t = t - 1.0 + e1 * jnp.exp(-t)
            t = t - 1.0 + e1 * jnp.exp(-t)
            return jnp.maximum(x, 0.0) + t

        idx_cps = []
        for c in range(nchunk):
            bc = base0 + c * chunk
            idx_cps.append(pltpu.async_copy(
                nodes_hbm.at[pl.ds(bc, chunk)],
                idx_all.at[c, pl.ds(0, chunk)], sem_i))
            idx_cps.append(pltpu.async_copy(
                pos_hbm.at[pl.ds(bc, chunk)],
                idx_all.at[c, pl.ds(chunk, chunk)], sem_i))
            idx_cps.append(pltpu.async_copy(
                neg_hbm.at[pl.ds(bc, chunk), :], negraw.at[c], sem_i))
        for cp in idx_cps:
            cp.wait()

        for c in range(nchunk):
            for j in range(chunk * K // L):
                f = j * L + iot
                vals = plsc.load_gather(negraw, [jnp.full((L,), c, jnp.int32),
                                                f // K, f % K])
                idx_all[c, pl.ds(2 * chunk + j * L, L)] = vals

        streams = [None] * nchunk
        streams[0] = pltpu.async_copy(
            table_hbm.at[idx_all.at[0]], bufs[0], ssems[0])

        zero = jnp.zeros((L,), jnp.float32)
        pp, nn = zero, zero
        for c in range(nchunk):
            if c + 1 < nchunk:
                streams[c + 1] = pltpu.async_copy(
                    table_hbm.at[idx_all.at[c + 1]],
                    bufs[(c + 1) % 2], ssems[(c + 1) % 2])
            streams[c].wait()
            rows = bufs[c % 2]

            def group_body(g, carry, rows=rows):
                pp, nn = carry
                rb = g * L + iot
                rp = rb + chunk
                rg = 2 * chunk + rb * K
                accs = [zero] * (1 + K)

                def d_body(d4, accs):
                    accs = list(accs)
                    for dd in range(4):
                        col = jnp.bitwise_and(iot + (d4 * 4 + dd), D - 1)
                        nd = plsc.load_gather(rows, [rb, col])
                        pd = plsc.load_gather(rows, [rp, col])
                        accs[0] = accs[0] + nd * pd
                        for kk in range(K):
                            gd = plsc.load_gather(rows, [rg + kk, col])
                            accs[1 + kk] = accs[1 + kk] + nd * gd
                    return tuple(accs)

                accs = lax.fori_loop(0, D // 4, d_body, tuple(accs))
                pp = pp + softplus(-accs[0])
                for kk in range(K):
                    nn = nn + softplus(accs[1 + kk])
                return pp, nn

            pp, nn = lax.fori_loop(0, chunk // L, group_body, (pp, nn))

        partials[0, :] = pp
        partials[1, :] = nn
        pltpu.sync_copy(partials, out_hbm.at[:, pl.ds(wid * L, L)])

    return k


def _tc_losses(batch: int):
    def body(s_ref, pos_out, neg_out):
        s = s_ref[...]
        pos_out[0, 0] = jnp.sum(s[0:1, :]) / batch
        neg_out[0, 0] = jnp.sum(s[1:2, :]) / (batch * K)

    return pl.pallas_call(
        body,
        out_shape=[
            jax.ShapeDtypeStruct((1, 1), jnp.float32),
            jax.ShapeDtypeStruct((1, 1), jnp.float32),
        ],
        out_specs=[
            pl.BlockSpec(memory_space=pltpu.SMEM),
            pl.BlockSpec(memory_space=pltpu.SMEM),
        ],
    )


def kernel(nodes, pos_neighbors, neg_neighbors, node_embeddings):
    batch = nodes.shape[0]
    table_p = jnp.pad(node_embeddings, ((0, 0), (0, DP - D)))
    partials = _sc_scores(batch)(
        nodes, pos_neighbors, neg_neighbors, table_p)
    pos_loss, neg_loss = _tc_losses(batch)(partials)
    return (pos_loss[0, 0], neg_loss[0, 0])


# speedup vs baseline: 1.9123x; 1.0113x over previous
import functools

import jax
import jax.numpy as jnp
from jax import lax
from jax.experimental import pallas as pl
from jax.experimental.pallas import tpu as pltpu
from jax.experimental.pallas import tpu_sc as plsc

D = 64
DP = 128
K = 5
L = 16

_info = plsc.get_sparse_core_info()
NC, NS = _info.num_cores, _info.num_subcores
NW = NC * NS


def _sc_scores(batch: int):
    bpw = batch // NW
    chunk = 32
    nchunk = bpw // chunk
    rows_per_chunk = chunk * (2 + K)
    mesh = plsc.VectorSubcoreMesh(core_axis_name="c", subcore_axis_name="s")

    @functools.partial(
        pl.kernel,
        out_type=jax.ShapeDtypeStruct((2, NW * L), jnp.float32),
        mesh=mesh,
        compiler_params=pltpu.CompilerParams(
            needs_layout_passes=False, use_tc_tiling_on_sc=False
        ),
        scratch_types=[
            pltpu.VMEM((nchunk, rows_per_chunk), jnp.int32),
            pltpu.VMEM((nchunk, chunk, K), jnp.int32),
            pltpu.VMEM((rows_per_chunk, DP), jnp.float32),
            pltpu.VMEM((rows_per_chunk, DP), jnp.float32),
            pltpu.VMEM((rows_per_chunk, DP), jnp.float32),
            pltpu.VMEM((2, L), jnp.float32),
            pltpu.SemaphoreType.DMA,
            pltpu.SemaphoreType.DMA,
            pltpu.SemaphoreType.DMA,
            pltpu.SemaphoreType.DMA,
        ],
    )
    def k(nodes_hbm, pos_hbm, neg_hbm, table_hbm, out_hbm,
          idx_all, negraw, rows_a, rows_b, rows_c, partials,
          sem_i, sem_0, sem_1, sem_2):
        wid = lax.axis_index("s") * NC + lax.axis_index("c")
        base0 = wid * bpw
        iot = lax.iota(jnp.int32, L)
        bufs = (rows_a, rows_b, rows_c)
        ssems = (sem_0, sem_1, sem_2)

        def softplus(x):
            e = jnp.exp(-jnp.abs(x))
            e1 = 1.0 + e
            t = e * (6.0 + e) / (6.0 + 4.0 * e)
            t = t - 1.0 + e1 * jnp.exp(-t)
            t = t - 1.0 + e1 * jnp.exp(-t)
            return jnp.maximum(x, 0.0) + t

        idx_cps = []
        for c in range(nchunk):
            bc = base0 + c * chunk
            idx_cps.append(pltpu.async_copy(
                nodes_hbm.at[pl.ds(bc, chunk)],
                idx_all.at[c, pl.ds(0, chunk)], sem_i))
            idx_cps.append(pltpu.async_copy(
                pos_hbm.at[pl.ds(bc, chunk)],
                idx_all.at[c, pl.ds(chunk, chunk)], sem_i))
            idx_cps.append(pltpu.async_copy(
                neg_hbm.at[pl.ds(bc, chunk), :], negraw.at[c], sem_i))
        for cp in idx_cps:
            cp.wait()

        for c in range(nchunk):
            for j in range(chunk * K // L):
                f = j * L + iot
                vals = plsc.load_gather(negraw, [jnp.full((L,), c, jnp.int32),
                                                f // K, f % K])
                idx_all[c, pl.ds(2 * chunk + j * L, L)] = vals

        streams = [None] * nchunk
        streams[0] = pltpu.async_copy(
            table_hbm.at[idx_all.at[0]], bufs[0], ssems[0])
        if nchunk > 1:
            streams[1] = pltpu.async_copy(
                table_hbm.at[idx_all.at[1]], bufs[1], ssems[1])

        zero = jnp.zeros((L,), jnp.float32)
        pp, nn = zero, zero
        for c in range(nchunk):
            if c + 2 < nchunk:
                streams[c + 2] = pltpu.async_copy(
                    table_hbm.at[idx_all.at[c + 2]],
                    bufs[(c + 2) % 3], ssems[(c + 2) % 3])
            streams[c].wait()
            rows = bufs[c % 3]

            def group_body(g, carry, rows=rows):
                pp, nn = carry
                rb = g * L + iot
                rp = rb + chunk
                rg = 2 * chunk + rb * K
                accs = [zero] * (1 + K)

                def d_body(d4, accs):
                    accs = list(accs)
                    for dd in range(4):
                        col = jnp.bitwise_and(iot + (d4 * 4 + dd), D - 1)
                        nd = plsc.load_gather(rows, [rb, col])
                        pd = plsc.load_gather(rows, [rp, col])
                        accs[0] = accs[0] + nd * pd
                        for kk in range(K):
                            gd = plsc.load_gather(rows, [rg + kk, col])
                            accs[1 + kk] = accs[1 + kk] + nd * gd
                    return tuple(accs)

                accs = lax.fori_loop(0, D // 4, d_body, tuple(accs))
                pp = pp + softplus(-accs[0])
                for kk in range(K):
                    nn = nn + softplus(accs[1 + kk])
                return pp, nn

            pp, nn = lax.fori_loop(0, chunk // L, group_body, (pp, nn))

        partials[0, :] = pp
        partials[1, :] = nn
        pltpu.sync_copy(partials, out_hbm.at[:, pl.ds(wid * L, L)])

    return k


def _tc_losses(batch: int):
    def body(s_ref, pos_out, neg_out):
        s = s_ref[...]
        pos_out[0, 0] = jnp.sum(s[0:1, :]) / batch
        neg_out[0, 0] = jnp.sum(s[1:2, :]) / (batch * K)

    return pl.pallas_call(
        body,
        out_shape=[
            jax.ShapeDtypeStruct((1, 1), jnp.float32),
            jax.ShapeDtypeStruct((1, 1), jnp.float32),
        ],
        out_specs=[
            pl.BlockSpec(memory_space=pltpu.SMEM),
            pl.BlockSpec(memory_space=pltpu.SMEM),
        ],
    )


def kernel(nodes, pos_neighbors, neg_neighbors, node_embeddings):
    batch = nodes.shape[0]
    table_p = jnp.pad(node_embeddings, ((0, 0), (0, DP - D)))
    partials = _sc_scores(batch)(
        nodes, pos_neighbors, neg_neighbors, table_p)
    pos_loss, neg_loss = _tc_losses(batch)(partials)
    return (pos_loss[0, 0], neg_loss[0, 0])
